# baseline (speedup 1.0000x reference)
import functools

import jax
import jax.numpy as jnp
from jax import lax
from jax.experimental import pallas as pl
from jax.experimental.pallas import tpu as pltpu
from jax.experimental.pallas import tpu_sc as plsc

D = 64
WPR = D // 2
N_TABLES = 6
CHUNK = 256
GRP = 16


def _sc_gather_sum(n_tokens, n_rows):
    info = plsc.get_sparse_core_info()
    nc, ns = info.num_cores, info.num_subcores
    nw = nc * ns
    assert n_tokens % (nw * CHUNK) == 0
    n_per_w = n_tokens // nw
    n_chunks = n_per_w // CHUNK
    assert n_chunks % 2 == 0
    tab_words = (n_rows * (WPR + 1) + 15) // 16 * 16

    mesh = plsc.VectorSubcoreMesh(core_axis_name="c", subcore_axis_name="s")

    ROW = 2 * D

    scratch = (
        [pltpu.VMEM((tab_words,), jnp.int32)]
        + [pltpu.VMEM((N_TABLES, CHUNK), jnp.int32) for _ in range(2)]
        + [pltpu.VMEM((CHUNK * ROW,), jnp.float32) for _ in range(2)]
        + [pltpu.SemaphoreType.DMA, pltpu.SemaphoreType.DMA,
           pltpu.SemaphoreType.DMA, pltpu.SemaphoreType.DMA]
    )

    @functools.partial(
        pl.kernel,
        mesh=mesh,
        out_type=jax.ShapeDtypeStruct((n_tokens * ROW,), jnp.float32),
        scratch_types=scratch,
        compiler_params=pltpu.CompilerParams(needs_layout_passes=False),
    )
    def k(widx_hbm, tab_hbm, out_hbm, tab_v, idx0, idx1, acc0, acc1,
          sem_tab, sem_idx, sem_o0, sem_o1):
        idx_bufs = (idx0, idx1)
        acc_bufs = (acc0, acc1)
        out_sems = (sem_o0, sem_o1)
        wid = lax.axis_index("s") * nc + lax.axis_index("c")
        base = wid * n_per_w

        pltpu.async_copy(tab_hbm, tab_v, sem_tab).wait()
        lane = lax.iota(jnp.int32, GRP)
        tok_off = lane * WPR

        def fetch_idx(c, buf):
            start = base + c * CHUNK
            return pltpu.async_copy(
                widx_hbm.at[:, pl.ds(start, CHUNK)], idx_bufs[buf], sem_idx)

        def wait_idx(buf):
            pltpu.make_async_copy(
                widx_hbm.at[:, pl.ds(base, CHUNK)], idx_bufs[buf],
                sem_idx).wait()

        def chunk_compute(buf):
            idx_v, acc = idx_bufs[buf], acc_bufs[buf]

            def grp_body(g, carry):
                wv = [idx_v[t, pl.ds(g * GRP, GRP)] for t in range(N_TABLES)]
                row_base = (g * GRP + lane) * ROW

                def rot(w):
                    return (lane + w) & (WPR - 1)

                def gather6(w):
                    r = rot(w)
                    return r, [plsc.load_gather(tab_v, [wv[t] + r])
                               for t in range(N_TABLES)]

                def sum6(vs):
                    b = [plsc.bitcast(v, jnp.bfloat16) for v in vs]
                    return ((b[0] + b[1]) + (b[2] + b[3])) + (b[4] + b[5])

                def emit(r, vs):
                    s32 = plsc.bitcast(sum6(vs), jnp.int32)
                    lo = plsc.bitcast(
                        lax.shift_left(s32, jnp.int32(16)), jnp.float32)
                    hi = plsc.bitcast(
                        s32 & jnp.int32(-65536), jnp.float32)
                    col = row_base + (r + r)
                    plsc.store_scatter(acc, [col], lo)
                    plsc.store_scatter(acc, [col + 1], hi)

                prev_r, prev = gather6(0)
                for w in range(1, WPR):
                    cur_r, cur = gather6(w)
                    emit(prev_r, prev)
                    prev_r, prev = cur_r, cur
                emit(prev_r, prev)
                return carry

            lax.fori_loop(0, CHUNK // GRP, grp_body, 0)

        def store_out(c, buf):
            start = base + c * CHUNK
            return pltpu.async_copy(
                acc_bufs[buf],
                out_hbm.at[pl.ds(start * ROW, CHUNK * ROW)], out_sems[buf])

        def wait_out(buf):
            pltpu.make_async_copy(
                acc_bufs[buf],
                out_hbm.at[pl.ds(base * ROW, CHUNK * ROW)],
                out_sems[buf]).wait()

        fetch_idx(0, 0)

        def loop_body(i, carry):
            c0 = i * 2
            for b in range(2):
                c = c0 + b
                wait_idx(b)

                @pl.when(c + 1 < n_chunks)
                def _():
                    fetch_idx(c + 1, 1 - b)

                @pl.when(c >= 2)
                def _():
                    wait_out(b)

                chunk_compute(b)
                store_out(c, b)
            return carry

        lax.fori_loop(0, n_chunks // 2, loop_body, 0)
        wait_out(0)
        wait_out(1)

    return k


def _tc_finish_body(emb_ref, nf_ref, w_ref, b_ref, out_ref):
    out_ref[...] = (
        emb_ref[:, :D]
        + jnp.dot(nf_ref[...], w_ref[...], preferred_element_type=jnp.float32)
        + b_ref[...]
    )


def kernel(role_ids, namespace_ids, label_ids, path_ids, depth_ids, position_ids,
           numeric_features, role_table, namespace_table, label_table, path_table,
           depth_table, position_table, num_w, num_b):
    B, T = role_ids.shape
    n = B * T
    nf = numeric_features.shape[-1]

    tables = (role_table, namespace_table, label_table, path_table,
              depth_table, position_table)
    sizes = [t.shape[0] for t in tables]
    bases = []
    acc = 0
    for s in sizes:
        bases.append(acc)
        acc += s
    n_rows = acc

    big = jnp.concatenate(tables, axis=0).astype(jnp.bfloat16)
    tab_packed = jax.lax.bitcast_convert_type(
        big.reshape(n_rows, WPR, 2), jnp.int32)
    tab_words = n_rows * (WPR + 1)
    tab_words_pad = (tab_words + 15) // 16 * 16
    tab_packed = jnp.pad(tab_packed, ((0, 0), (0, 1))).reshape(tab_words)
    tab_packed = jnp.pad(tab_packed, (0, tab_words_pad - tab_words))

    ids = (role_ids, namespace_ids, label_ids, path_ids, depth_ids,
           position_ids)
    widx = jnp.stack([
        (i.reshape(n).astype(jnp.int32) + b) * (WPR + 1)
        for i, b in zip(ids, bases)
    ])

    half = n // 2
    sc = _sc_gather_sum(half, n_rows)
    nf_flat = numeric_features.reshape(n, nf)
    blk = 2048

    outs = []
    for h in range(2):
        emb = sc(widx[:, h * half:(h + 1) * half],
                 tab_packed).reshape(half, 2 * D)
        outs.append(pl.pallas_call(
            _tc_finish_body,
            grid=(half // blk,),
            in_specs=[
                pl.BlockSpec((blk, 2 * D), lambda i: (i, 0)),
                pl.BlockSpec((blk, nf), lambda i: (i, 0)),
                pl.BlockSpec((nf, D), lambda i: (0, 0)),
                pl.BlockSpec((1, D), lambda i: (0, 0)),
            ],
            out_specs=pl.BlockSpec((blk, D), lambda i: (i, 0)),
            out_shape=jax.ShapeDtypeStruct((half, D), jnp.float32),
        )(emb, nf_flat[h * half:(h + 1) * half], num_w,
          num_b.reshape(1, D)))

    return jnp.concatenate(outs, axis=0).reshape(B, T, D)


# speedup vs baseline: 1.1784x; 1.1784x over previous
import functools

import jax
import jax.numpy as jnp
from jax import lax
from jax.experimental import pallas as pl
from jax.experimental.pallas import tpu as pltpu
from jax.experimental.pallas import tpu_sc as plsc

D = 64
WPR = D // 2
N_TABLES = 6
CHUNK = 256
GRP = 16


def _sc_gather_sum(n_tokens, n_rows):
    info = plsc.get_sparse_core_info()
    nc, ns = info.num_cores, info.num_subcores
    nw = nc * ns
    assert n_tokens % (nw * CHUNK) == 0
    n_per_w = n_tokens // nw
    n_chunks = n_per_w // CHUNK
    assert n_chunks % 2 == 0
    tab_words = (n_rows * (WPR + 1) + 15) // 16 * 16

    mesh = plsc.VectorSubcoreMesh(core_axis_name="c", subcore_axis_name="s")

    ROW = 2 * D

    scratch = (
        [pltpu.VMEM((tab_words,), jnp.int32)]
        + [pltpu.VMEM((N_TABLES, CHUNK), jnp.int32) for _ in range(2)]
        + [pltpu.VMEM((CHUNK * ROW,), jnp.float32) for _ in range(2)]
        + [pltpu.SemaphoreType.DMA, pltpu.SemaphoreType.DMA,
           pltpu.SemaphoreType.DMA, pltpu.SemaphoreType.DMA]
    )

    @functools.partial(
        pl.kernel,
        mesh=mesh,
        out_type=jax.ShapeDtypeStruct((n_tokens * ROW,), jnp.float32),
        scratch_types=scratch,
        compiler_params=pltpu.CompilerParams(needs_layout_passes=False),
    )
    def k(widx_hbm, tab_hbm, out_hbm, tab_v, idx0, idx1, acc0, acc1,
          sem_tab, sem_idx, sem_o0, sem_o1):
        idx_bufs = (idx0, idx1)
        acc_bufs = (acc0, acc1)
        out_sems = (sem_o0, sem_o1)
        wid = lax.axis_index("s") * nc + lax.axis_index("c")
        base = wid * n_per_w

        pltpu.async_copy(tab_hbm, tab_v, sem_tab).wait()
        lane = lax.iota(jnp.int32, GRP)
        tok_off = lane * WPR

        def fetch_idx(c, buf):
            start = base + c * CHUNK
            return pltpu.async_copy(
                widx_hbm.at[:, pl.ds(start, CHUNK)], idx_bufs[buf], sem_idx)

        def wait_idx(buf):
            pltpu.make_async_copy(
                widx_hbm.at[:, pl.ds(base, CHUNK)], idx_bufs[buf],
                sem_idx).wait()

        def chunk_compute(buf):
            idx_v, acc = idx_bufs[buf], acc_bufs[buf]

            def grp_body(g, carry):
                wv = [idx_v[t, pl.ds(g * GRP, GRP)] for t in range(N_TABLES)]
                row_base = (g * GRP + lane) * ROW

                def rot(w):
                    return (lane + w) & (WPR - 1)

                def gather6(w):
                    r = rot(w)
                    return r, [plsc.load_gather(tab_v, [wv[t] + r])
                               for t in range(N_TABLES)]

                def sum6(vs):
                    b = [plsc.bitcast(v, jnp.bfloat16) for v in vs]
                    return ((b[0] + b[1]) + (b[2] + b[3])) + (b[4] + b[5])

                def emit(r, vs):
                    s32 = plsc.bitcast(sum6(vs), jnp.int32)
                    lo = plsc.bitcast(
                        lax.shift_left(s32, jnp.int32(16)), jnp.float32)
                    hi = plsc.bitcast(
                        s32 & jnp.int32(-65536), jnp.float32)
                    col = row_base + (r + r)
                    plsc.store_scatter(acc, [col], lo)
                    plsc.store_scatter(acc, [col + 1], hi)

                prev_r, prev = gather6(0)
                for w in range(1, WPR):
                    cur_r, cur = gather6(w)
                    emit(prev_r, prev)
                    prev_r, prev = cur_r, cur
                emit(prev_r, prev)
                return carry

            lax.fori_loop(0, CHUNK // GRP, grp_body, 0)

        def store_out(c, buf):
            start = base + c * CHUNK
            return pltpu.async_copy(
                acc_bufs[buf],
                out_hbm.at[pl.ds(start * ROW, CHUNK * ROW)], out_sems[buf])

        def wait_out(buf):
            pltpu.make_async_copy(
                acc_bufs[buf],
                out_hbm.at[pl.ds(base * ROW, CHUNK * ROW)],
                out_sems[buf]).wait()

        fetch_idx(0, 0)

        def loop_body(i, carry):
            c0 = i * 2
            for b in range(2):
                c = c0 + b
                wait_idx(b)

                @pl.when(c + 1 < n_chunks)
                def _():
                    fetch_idx(c + 1, 1 - b)

                @pl.when(c >= 2)
                def _():
                    wait_out(b)

                chunk_compute(b)
                store_out(c, b)
            return carry

        lax.fori_loop(0, n_chunks // 2, loop_body, 0)
        wait_out(0)
        wait_out(1)

    return k


def _tc_finish_body(emb_ref, nf_ref, w_ref, b_ref, out_ref):
    out_ref[...] = (
        emb_ref[:, :D]
        + jnp.dot(nf_ref[...], w_ref[...], preferred_element_type=jnp.float32)
        + b_ref[...]
    )


def kernel(role_ids, namespace_ids, label_ids, path_ids, depth_ids, position_ids,
           numeric_features, role_table, namespace_table, label_table, path_table,
           depth_table, position_table, num_w, num_b):
    B, T = role_ids.shape
    n = B * T
    nf = numeric_features.shape[-1]

    tables = (role_table, namespace_table, label_table, path_table,
              depth_table, position_table)
    sizes = [t.shape[0] for t in tables]
    bases = []
    acc = 0
    for s in sizes:
        bases.append(acc)
        acc += s
    n_rows = acc

    big = jnp.concatenate(tables, axis=0).astype(jnp.bfloat16)
    tab_packed = jax.lax.bitcast_convert_type(
        big.reshape(n_rows, WPR, 2), jnp.int32)
    tab_words = n_rows * (WPR + 1)
    tab_words_pad = (tab_words + 15) // 16 * 16
    tab_packed = jnp.pad(tab_packed, ((0, 0), (0, 1))).reshape(tab_words)
    tab_packed = jnp.pad(tab_packed, (0, tab_words_pad - tab_words))

    ids = (role_ids, namespace_ids, label_ids, path_ids, depth_ids,
           position_ids)
    widx = jnp.stack([
        (i.reshape(n).astype(jnp.int32) + b) * (WPR + 1)
        for i, b in zip(ids, bases)
    ])

    sc = _sc_gather_sum(n, n_rows)
    emb = sc(widx, tab_packed).reshape(n, 2 * D)

    blk = 4096
    out = pl.pallas_call(
        _tc_finish_body,
        grid=(n // blk,),
        in_specs=[
            pl.BlockSpec((blk, 2 * D), lambda i: (i, 0)),
            pl.BlockSpec((blk, nf), lambda i: (i, 0)),
            pl.BlockSpec((nf, D), lambda i: (0, 0)),
            pl.BlockSpec((1, D), lambda i: (0, 0)),
        ],
        out_specs=pl.BlockSpec((blk, D), lambda i: (i, 0)),
        out_shape=jax.ShapeDtypeStruct((n, D), jnp.float32),
    )(emb, numeric_features.reshape(n, nf), num_w, num_b.reshape(1, D))

    return out.reshape(B, T, D)


# speedup vs baseline: 1.1959x; 1.0149x over previous
import functools

import jax
import jax.numpy as jnp
from jax import lax
from jax.experimental import pallas as pl
from jax.experimental.pallas import tpu as pltpu
from jax.experimental.pallas import tpu_sc as plsc

D = 64
WPR = D // 2
N_TABLES = 6
CHUNK = 256
GRP = 16


def _sc_gather_sum(n_tokens, n_rows):
    info = plsc.get_sparse_core_info()
    nc, ns = info.num_cores, info.num_subcores
    nw = nc * ns
    assert n_tokens % (nw * CHUNK) == 0
    n_per_w = n_tokens // nw
    n_chunks = n_per_w // CHUNK
    assert n_chunks % 2 == 0
    tab_words = (n_rows * (WPR + 1) + 15) // 16 * 16

    mesh = plsc.VectorSubcoreMesh(core_axis_name="c", subcore_axis_name="s")

    ROW = 2 * D

    scratch = (
        [pltpu.VMEM((tab_words,), jnp.int32)]
        + [pltpu.VMEM((N_TABLES, CHUNK), jnp.int32) for _ in range(2)]
        + [pltpu.VMEM((CHUNK * ROW,), jnp.float32) for _ in range(2)]
        + [pltpu.SemaphoreType.DMA, pltpu.SemaphoreType.DMA,
           pltpu.SemaphoreType.DMA, pltpu.SemaphoreType.DMA]
    )

    @functools.partial(
        pl.kernel,
        mesh=mesh,
        out_type=jax.ShapeDtypeStruct((n_tokens * ROW,), jnp.float32),
        scratch_types=scratch,
        compiler_params=pltpu.CompilerParams(needs_layout_passes=False),
    )
    def k(widx_hbm, tab_hbm, out_hbm, tab_v, idx0, idx1, acc0, acc1,
          sem_tab, sem_idx, sem_o0, sem_o1):
        idx_bufs = (idx0, idx1)
        acc_bufs = (acc0, acc1)
        out_sems = (sem_o0, sem_o1)
        wid = lax.axis_index("s") * nc + lax.axis_index("c")
        base = wid * n_per_w

        pltpu.async_copy(tab_hbm, tab_v, sem_tab).wait()
        lane = lax.iota(jnp.int32, GRP)
        tok_off = lane * WPR

        def fetch_idx(c, buf):
            start = base + c * CHUNK
            return pltpu.async_copy(
                widx_hbm.at[:, pl.ds(start, CHUNK)], idx_bufs[buf], sem_idx)

        def wait_idx(buf):
            pltpu.make_async_copy(
                widx_hbm.at[:, pl.ds(base, CHUNK)], idx_bufs[buf],
                sem_idx).wait()

        def chunk_compute(buf):
            idx_v, acc = idx_bufs[buf], acc_bufs[buf]

            def grp_body(g, carry):
                wv = [idx_v[t, pl.ds(g * GRP, GRP)] for t in range(N_TABLES)]
                row_base = (g * GRP + lane) * ROW

                def rot(w):
                    return (lane + w) & (WPR - 1)

                def gather6(w):
                    r = rot(w)
                    return r, [plsc.load_gather(tab_v, [wv[t] + r])
                               for t in range(N_TABLES)]

                def sum6(vs):
                    b = [plsc.bitcast(v, jnp.bfloat16) for v in vs]
                    return ((b[0] + b[1]) + (b[2] + b[3])) + (b[4] + b[5])

                def emit(r, vs):
                    s32 = plsc.bitcast(sum6(vs), jnp.int32)
                    lo = plsc.bitcast(
                        lax.shift_left(s32, jnp.int32(16)), jnp.float32)
                    hi = plsc.bitcast(
                        s32 & jnp.int32(-65536), jnp.float32)
                    col = row_base + (r + r)
                    plsc.store_scatter(acc, [col], lo)
                    plsc.store_scatter(acc, [col + 1], hi)

                prev_r, prev = gather6(0)
                for w in range(1, WPR):
                    cur_r, cur = gather6(w)
                    emit(prev_r, prev)
                    prev_r, prev = cur_r, cur
                emit(prev_r, prev)
                return carry

            lax.fori_loop(0, CHUNK // GRP, grp_body, 0)

        def store_out(c, buf):
            start = base + c * CHUNK
            return pltpu.async_copy(
                acc_bufs[buf],
                out_hbm.at[pl.ds(start * ROW, CHUNK * ROW)], out_sems[buf])

        def wait_out(buf):
            pltpu.make_async_copy(
                acc_bufs[buf],
                out_hbm.at[pl.ds(base * ROW, CHUNK * ROW)],
                out_sems[buf]).wait()

        fetch_idx(0, 0)

        def loop_body(i, carry):
            c0 = i * 2
            for b in range(2):
                c = c0 + b
                wait_idx(b)

                @pl.when(c + 1 < n_chunks)
                def _():
                    fetch_idx(c + 1, 1 - b)

                @pl.when(c >= 2)
                def _():
                    wait_out(b)

                chunk_compute(b)
                store_out(c, b)
            return carry

        lax.fori_loop(0, n_chunks // 2, loop_body, 0)
        wait_out(0)
        wait_out(1)

    return k


def _tc_finish_body(emb_ref, nf_ref, w_ref, b_ref, out_ref):
    out_ref[...] = (
        emb_ref[:, :D]
        + jnp.dot(nf_ref[...], w_ref[...], preferred_element_type=jnp.float32)
        + b_ref[...]
    )


def kernel(role_ids, namespace_ids, label_ids, path_ids, depth_ids, position_ids,
           numeric_features, role_table, namespace_table, label_table, path_table,
           depth_table, position_table, num_w, num_b):
    B, T = role_ids.shape
    n = B * T
    nf = numeric_features.shape[-1]

    tables = (role_table, namespace_table, label_table, path_table,
              depth_table, position_table)
    sizes = [t.shape[0] for t in tables]
    bases = []
    acc = 0
    for s in sizes:
        bases.append(acc)
        acc += s
    n_rows = acc

    big = jnp.concatenate(tables, axis=0).astype(jnp.bfloat16)
    tab_packed = jax.lax.bitcast_convert_type(
        big.reshape(n_rows, WPR, 2), jnp.int32)
    tab_words = n_rows * (WPR + 1)
    tab_words_pad = (tab_words + 15) // 16 * 16
    tab_packed = jnp.pad(tab_packed, ((0, 0), (0, 1))).reshape(tab_words)
    tab_packed = jnp.pad(tab_packed, (0, tab_words_pad - tab_words))

    ids = (role_ids, namespace_ids, label_ids, path_ids, depth_ids,
           position_ids)
    widx = jnp.stack([
        (i.reshape(n).astype(jnp.int32) + b) * (WPR + 1)
        for i, b in zip(ids, bases)
    ])

    sc = _sc_gather_sum(n, n_rows)
    emb = sc(widx, tab_packed).reshape(n, 2 * D)

    blk = 8192
    out = pl.pallas_call(
        _tc_finish_body,
        grid=(n // blk,),
        in_specs=[
            pl.BlockSpec((blk, 2 * D), lambda i: (i, 0)),
            pl.BlockSpec((blk, nf), lambda i: (i, 0)),
            pl.BlockSpec((nf, D), lambda i: (0, 0)),
            pl.BlockSpec((1, D), lambda i: (0, 0)),
        ],
        out_specs=pl.BlockSpec((blk, D), lambda i: (i, 0)),
        out_shape=jax.ShapeDtypeStruct((n, D), jnp.float32),
    )(emb, numeric_features.reshape(n, nf), num_w, num_b.reshape(1, D))

    return out.reshape(B, T, D)


# speedup vs baseline: 1.1961x; 1.0002x over previous
import functools

import jax
import jax.numpy as jnp
from jax import lax
from jax.experimental import pallas as pl
from jax.experimental.pallas import tpu as pltpu
from jax.experimental.pallas import tpu_sc as plsc

D = 64
WPR = D // 2
N_TABLES = 6
CHUNK = 256
GRP = 16


def _sc_gather_sum(n_tokens, n_rows):
    info = plsc.get_sparse_core_info()
    nc, ns = info.num_cores, info.num_subcores
    nw = nc * ns
    assert n_tokens % (nw * CHUNK) == 0
    n_per_w = n_tokens // nw
    n_chunks = n_per_w // CHUNK
    assert n_chunks % 2 == 0
    tab_words = (n_rows * (WPR + 1) + 15) // 16 * 16

    mesh = plsc.VectorSubcoreMesh(core_axis_name="c", subcore_axis_name="s")

    ROW = 2 * D

    scratch = (
        [pltpu.VMEM((tab_words,), jnp.int32)]
        + [pltpu.VMEM((N_TABLES, CHUNK), jnp.int32) for _ in range(2)]
        + [pltpu.VMEM((CHUNK * ROW,), jnp.float32) for _ in range(2)]
        + [pltpu.SemaphoreType.DMA, pltpu.SemaphoreType.DMA,
           pltpu.SemaphoreType.DMA, pltpu.SemaphoreType.DMA]
    )

    @functools.partial(
        pl.kernel,
        mesh=mesh,
        out_type=jax.ShapeDtypeStruct((n_tokens * ROW,), jnp.float32),
        scratch_types=scratch,
        compiler_params=pltpu.CompilerParams(needs_layout_passes=False),
    )
    def k(widx_hbm, tab_hbm, out_hbm, tab_v, idx0, idx1, acc0, acc1,
          sem_tab, sem_idx, sem_o0, sem_o1):
        idx_bufs = (idx0, idx1)
        acc_bufs = (acc0, acc1)
        out_sems = (sem_o0, sem_o1)
        wid = lax.axis_index("s") * nc + lax.axis_index("c")
        base = wid * n_per_w

        pltpu.async_copy(tab_hbm, tab_v, sem_tab).wait()
        lane = lax.iota(jnp.int32, GRP)
        tok_off = lane * WPR

        def fetch_idx(c, buf):
            start = base + c * CHUNK
            return pltpu.async_copy(
                widx_hbm.at[:, pl.ds(start, CHUNK)], idx_bufs[buf], sem_idx)

        def wait_idx(buf):
            pltpu.make_async_copy(
                widx_hbm.at[:, pl.ds(base, CHUNK)], idx_bufs[buf],
                sem_idx).wait()

        def chunk_compute(buf):
            idx_v, acc = idx_bufs[buf], acc_bufs[buf]

            def grp_body(g, carry):
                wv = [idx_v[t, pl.ds(g * GRP, GRP)] for t in range(N_TABLES)]
                row_base = (g * GRP + lane) * ROW

                def rot(w):
                    return (lane + w) & (WPR - 1)

                def gather6(w):
                    r = rot(w)
                    return r, [plsc.load_gather(tab_v, [wv[t] + r])
                               for t in range(N_TABLES)]

                def sum6(vs):
                    b = [plsc.bitcast(v, jnp.bfloat16) for v in vs]
                    return ((b[0] + b[1]) + (b[2] + b[3])) + (b[4] + b[5])

                def emit(r, vs):
                    s32 = plsc.bitcast(sum6(vs), jnp.int32)
                    lo = plsc.bitcast(
                        lax.shift_left(s32, jnp.int32(16)), jnp.float32)
                    hi = plsc.bitcast(
                        s32 & jnp.int32(-65536), jnp.float32)
                    col = row_base + (r + r)
                    plsc.store_scatter(acc, [col], lo)
                    plsc.store_scatter(acc, [col + 1], hi)

                prev_r, prev = gather6(0)
                for w in range(1, WPR):
                    cur_r, cur = gather6(w)
                    emit(prev_r, prev)
                    prev_r, prev = cur_r, cur
                emit(prev_r, prev)
                return carry

            lax.fori_loop(0, CHUNK // GRP, grp_body, 0)

        def store_out(c, buf):
            start = base + c * CHUNK
            return pltpu.async_copy(
                acc_bufs[buf],
                out_hbm.at[pl.ds(start * ROW, CHUNK * ROW)], out_sems[buf])

        def wait_out(buf):
            pltpu.make_async_copy(
                acc_bufs[buf],
                out_hbm.at[pl.ds(base * ROW, CHUNK * ROW)],
                out_sems[buf]).wait()

        fetch_idx(0, 0)

        def loop_body(i, carry):
            c0 = i * 2
            for b in range(2):
                c = c0 + b
                wait_idx(b)

                @pl.when(c + 1 < n_chunks)
                def _():
                    fetch_idx(c + 1, 1 - b)

                @pl.when(c >= 2)
                def _():
                    wait_out(b)

                chunk_compute(b)
                store_out(c, b)
            return carry

        lax.fori_loop(0, n_chunks // 2, loop_body, 0)
        wait_out(0)
        wait_out(1)

    return k


def _tc_finish_body(emb_ref, nf_ref, w_ref, b_ref, out_ref):
    out_ref[...] = (
        emb_ref[:, :D]
        + jnp.dot(nf_ref[...], w_ref[...], preferred_element_type=jnp.float32)
        + b_ref[...]
    )


def kernel(role_ids, namespace_ids, label_ids, path_ids, depth_ids, position_ids,
           numeric_features, role_table, namespace_table, label_table, path_table,
           depth_table, position_table, num_w, num_b):
    B, T = role_ids.shape
    n = B * T
    nf = numeric_features.shape[-1]

    tables = (role_table, namespace_table, label_table, path_table,
              depth_table, position_table)
    sizes = [t.shape[0] for t in tables]
    bases = []
    acc = 0
    for s in sizes:
        bases.append(acc)
        acc += s
    n_rows = acc

    big = jnp.concatenate(tables, axis=0).astype(jnp.bfloat16)
    tab_packed = jax.lax.bitcast_convert_type(
        big.reshape(n_rows, WPR, 2), jnp.int32)
    tab_words = n_rows * (WPR + 1)
    tab_words_pad = (tab_words + 15) // 16 * 16
    tab_packed = jnp.pad(tab_packed, ((0, 0), (0, 1))).reshape(tab_words)
    tab_packed = jnp.pad(tab_packed, (0, tab_words_pad - tab_words))

    ids = (role_ids, namespace_ids, label_ids, path_ids, depth_ids,
           position_ids)
    widx = jnp.stack([
        (i.reshape(n).astype(jnp.int32) + b) * (WPR + 1)
        for i, b in zip(ids, bases)
    ])

    sc = _sc_gather_sum(n, n_rows)
    emb = sc(widx, tab_packed).reshape(n, 2 * D)

    blk = 16384
    out = pl.pallas_call(
        _tc_finish_body,
        grid=(n // blk,),
        in_specs=[
            pl.BlockSpec((blk, 2 * D), lambda i: (i, 0)),
            pl.BlockSpec((blk, nf), lambda i: (i, 0)),
            pl.BlockSpec((nf, D), lambda i: (0, 0)),
            pl.BlockSpec((1, D), lambda i: (0, 0)),
        ],
        out_specs=pl.BlockSpec((blk, D), lambda i: (i, 0)),
        out_shape=jax.ShapeDtypeStruct((n, D), jnp.float32),
    )(emb, numeric_features.reshape(n, nf), num_w, num_b.reshape(1, D))

    return out.reshape(B, T, D)
